# baseline (speedup 1.0000x reference)
import functools
import math

import jax
import jax.numpy as jnp
from jax import lax
from jax.experimental import pallas as pl
from jax.experimental.pallas import tpu as pltpu
from jax.experimental.pallas import tpu_sc as plsc

NC = 2
NS = 16
NW = NC * NS

SUB = 100
KSUB = 8
C = SUB * KSUB


def _sc_embed(n_rows, d, seq):
  chunks = n_rows // (NW * C)
  assert chunks * NW * C == n_rows
  mesh = plsc.VectorSubcoreMesh(core_axis_name="c", subcore_axis_name="s")

  @functools.partial(
      pl.kernel,
      out_type=jax.ShapeDtypeStruct((n_rows, d), jnp.float32),
      mesh=mesh,
      scratch_types=[
          pltpu.VMEM((KSUB, SUB), jnp.int32),
          pltpu.VMEM((C, d), jnp.float32),
          pltpu.VMEM((seq, d), jnp.float32),
          pltpu.SemaphoreType.DMA,
      ],
      compiler_params=pltpu.CompilerParams(use_tc_tiling_on_sc=False),
  )
  def body(x_hbm, pe_hbm, w_hbm, out_hbm, idx_v, rows_v, pe_v, gsem):
    wid = lax.axis_index("s") * NC + lax.axis_index("c")
    scale = float(math.sqrt(d))

    pltpu.sync_copy(pe_hbm, pe_v)

    @pl.loop(0, chunks)
    def chunk_loop(g):
      row0 = pl.multiple_of((wid * chunks + g) * C, C)

      pltpu.sync_copy(x_hbm.at[pl.ds(pl.multiple_of(row0 // SUB, KSUB), KSUB)], idx_v)

      cps = [
          pltpu.async_copy(
              w_hbm.at[idx_v.at[k]],
              rows_v.at[pl.ds(k * SUB, SUB)],
              gsem,
          )
          for k in range(KSUB)
      ]
      for cp in cps:
        cp.wait()

      for t in range(C // seq):
        @pl.loop(0, seq)
        def row_loop(r, t=t):
          for j in range(d // 16):
            sl = pl.ds(j * 16, 16)
            rr = t * seq + r
            rows_v[rr, sl] = rows_v[rr, sl] * scale + pe_v[r, sl]

      pltpu.sync_copy(rows_v, out_hbm.at[pl.ds(row0, C)])

  return body


def kernel(x, weight, pe, timestep):
  batch, seq = x.shape
  d = weight.shape[1]
  n_rows = batch * seq
  pe_sl = lax.dynamic_slice_in_dim(pe, timestep, seq, axis=0)
  x2d = x.reshape(n_rows // SUB, SUB).astype(jnp.int32)
  out = _sc_embed(n_rows, d, seq)(x2d, pe_sl, weight)
  return out.reshape(batch, seq, d)
